# baseline (speedup 1.0000x reference)
import functools

import jax
import jax.numpy as jnp
from jax import lax
from jax.experimental import pallas as pl
from jax.experimental.pallas import tpu as pltpu
from jax.experimental.pallas import tpu_sc as plsc

N = 10000
E = 320000
K = 20000
H = 32
B = 128
NTILES = 32
NSUB = 16
CH = 80
EPT = CH * B
EPAD = NTILES * EPT
KBUF = 8
NROWS = 10112
RPS = NROWS // NSUB


def _sc_mesh():
    return plsc.VectorSubcoreMesh(core_axis_name="c", subcore_axis_name="s")


def _deg_call(dst3, zcol, ones_col):
    @functools.partial(
        pl.kernel,
        out_type=jax.ShapeDtypeStruct((2, NROWS, 16), jnp.float32),
        mesh=_sc_mesh(),
        scratch_types=[
            pltpu.VMEM((CH, B), jnp.int32),
            pltpu.VMEM((B, 16), jnp.float32),
            pltpu.VMEM_SHARED((NROWS, 16), jnp.float32),
        ],
        compiler_params=pltpu.CompilerParams(use_tc_tiling_on_sc=False),
    )
    def k(dst_hbm, z_hbm, ones_hbm, out_hbm, dst_all, ones_v, acc_sh):
        c = lax.axis_index("c")
        s = lax.axis_index("s")
        wid = c * NSUB + s
        pltpu.sync_copy(z_hbm.at[pl.ds(s * RPS, RPS)],
                        acc_sh.at[pl.ds(s * RPS, RPS)])
        pltpu.sync_copy(ones_hbm, ones_v)
        pltpu.sync_copy(dst_hbm.at[wid], dst_all)
        plsc.subcore_barrier()

        def body(g, _):
            pltpu.sync_copy(ones_v, acc_sh.at[dst_all.at[g]], add=True)
            return ()

        lax.fori_loop(0, CH, body, ())
        plsc.subcore_barrier()
        pltpu.sync_copy(acc_sh.at[pl.ds(s * RPS, RPS)],
                        out_hbm.at[c, pl.ds(s * RPS, RPS)])

    return k(dst3, zcol, ones_col)


def _msg_call(hwp, src3, dst3, z32):
    @functools.partial(
        pl.kernel,
        out_type=jax.ShapeDtypeStruct((4, NROWS, H), jnp.float32),
        mesh=_sc_mesh(),
        scratch_types=[
            pltpu.VMEM((CH, B), jnp.int32),
            pltpu.VMEM((CH, B), jnp.int32),
            [pltpu.VMEM((B, H), jnp.float32) for _ in range(KBUF)],
            [pltpu.VMEM_SHARED((NROWS, H), jnp.float32) for _ in range(2)],
            [pltpu.SemaphoreType.DMA for _ in range(KBUF)],
        ],
        compiler_params=pltpu.CompilerParams(use_tc_tiling_on_sc=False),
    )
    def k(hwp_hbm, src_hbm, dst_hbm, z_hbm, out_hbm,
          src_all, dst_all, bufs, accs, sems):
        c = lax.axis_index("c")
        s = lax.axis_index("s")
        wid = c * NSUB + s
        half = s // 8
        pltpu.sync_copy(z_hbm.at[pl.ds(s * RPS, RPS)],
                        accs[0].at[pl.ds(s * RPS, RPS)])
        pltpu.sync_copy(z_hbm.at[pl.ds(s * RPS, RPS)],
                        accs[1].at[pl.ds(s * RPS, RPS)])
        pltpu.sync_copy(src_hbm.at[wid], src_all)
        pltpu.sync_copy(dst_hbm.at[wid], dst_all)
        plsc.subcore_barrier()

        def do_group(j, acc_sh):
            g0 = j * KBUF
            ds = [
                pltpu.async_copy(hwp_hbm.at[src_all.at[g0 + b]],
                                 bufs[b], sems[b])
                for b in range(KBUF)
            ]
            for b in range(KBUF):
                ds[b].wait()
                pltpu.sync_copy(bufs[b], acc_sh.at[dst_all.at[g0 + b]],
                                add=True)

        def body0(j, _):
            do_group(j, accs[0])
            return ()

        def body1(j, _):
            do_group(j, accs[1])
            return ()

        @pl.when(half == 0)
        def _():
            lax.fori_loop(0, CH // KBUF, body0, ())

        @pl.when(half == 1)
        def _():
            lax.fori_loop(0, CH // KBUF, body1, ())

        plsc.subcore_barrier()
        pltpu.sync_copy(accs[0].at[pl.ds(s * RPS, RPS)],
                        out_hbm.at[2 * c, pl.ds(s * RPS, RPS)])
        pltpu.sync_copy(accs[1].at[pl.ds(s * RPS, RPS)],
                        out_hbm.at[2 * c + 1, pl.ds(s * RPS, RPS)])

    return k(hwp, src3, dst3, z32)


def _embed_call(S, Rp, xp, We1p, be1p, We2p, be2p, Wg1p):
    BN = 80
    grid = (N // BN,)

    def body(s_ref, rp_ref, xp_ref, w1_ref, b1_ref, w2_ref, b2_ref,
             wg_ref, hw_ref):
        sub = jnp.dot(s_ref[...], rp_ref[...],
                      preferred_element_type=jnp.float32)
        h = xp_ref[...] + sub
        t = jnp.tanh(jnp.dot(h, w1_ref[...],
                             preferred_element_type=jnp.float32) + b1_ref[...])
        h2 = jnp.tanh(jnp.dot(t, w2_ref[...],
                              preferred_element_type=jnp.float32) + b2_ref[...])
        hw1 = jnp.dot(h2, wg_ref[...], preferred_element_type=jnp.float32)
        hw_ref[...] = hw1[:, :H]

    return pl.pallas_call(
        body,
        grid=grid,
        in_specs=[
            pl.BlockSpec((BN, K), lambda i: (i, 0)),
            pl.BlockSpec((K, 128), lambda i: (0, 0)),
            pl.BlockSpec((BN, 128), lambda i: (i, 0)),
            pl.BlockSpec((128, 128), lambda i: (0, 0)),
            pl.BlockSpec((1, 128), lambda i: (0, 0)),
            pl.BlockSpec((128, 128), lambda i: (0, 0)),
            pl.BlockSpec((1, 128), lambda i: (0, 0)),
            pl.BlockSpec((128, 128), lambda i: (0, 0)),
        ],
        out_specs=pl.BlockSpec((BN, H), lambda i: (i, 0)),
        out_shape=jax.ShapeDtypeStruct((N, H), jnp.float32),
    )(S, Rp, xp, We1p, be1p, We2p, be2p, Wg1p)


def _scale_call(hw1, degp):
    BN = 2000
    grid = (N // BN,)

    def body(hw_ref, dg_ref, hwp_ref, dinv_ref):
        deg = dg_ref[0] + dg_ref[1] + 1.0
        dinv = lax.rsqrt(deg)
        hwp_ref[...] = hw_ref[...] * dinv
        dinv_ref[...] = dinv

    return pl.pallas_call(
        body,
        grid=grid,
        in_specs=[
            pl.BlockSpec((BN, H), lambda i: (i, 0)),
            pl.BlockSpec((2, BN, 1), lambda i: (0, i, 0)),
        ],
        out_specs=[
            pl.BlockSpec((BN, H), lambda i: (i, 0)),
            pl.BlockSpec((BN, 1), lambda i: (i, 0)),
        ],
        out_shape=[
            jax.ShapeDtypeStruct((N, H), jnp.float32),
            jax.ShapeDtypeStruct((N, 1), jnp.float32),
        ],
    )(hw1, degp)


def _layer_call(accp, hwp, dinv, bg, Wnextp):
    BN = 2000
    grid = (N // BN,)

    def body(a_ref, hw_ref, dv_ref, bg_ref, wn_ref, out_ref):
        acc = a_ref[0] + a_ref[1] + a_ref[2] + a_ref[3] + hw_ref[...]
        g = jnp.maximum(acc * dv_ref[...] + bg_ref[...], 0.0)
        gp = jnp.concatenate(
            [g, jnp.zeros((BN, 128 - H), jnp.float32)], axis=1)
        hw2 = jnp.dot(gp, wn_ref[...], preferred_element_type=jnp.float32)
        out_ref[...] = (hw2 * dv_ref[...])[:, :H]

    return pl.pallas_call(
        body,
        grid=grid,
        in_specs=[
            pl.BlockSpec((4, BN, H), lambda i: (0, i, 0)),
            pl.BlockSpec((BN, H), lambda i: (i, 0)),
            pl.BlockSpec((BN, 1), lambda i: (i, 0)),
            pl.BlockSpec((1, H), lambda i: (0, 0)),
            pl.BlockSpec((128, 128), lambda i: (0, 0)),
        ],
        out_specs=pl.BlockSpec((BN, H), lambda i: (i, 0)),
        out_shape=jax.ShapeDtypeStruct((N, H), jnp.float32),
    )(accp, hwp, dinv, bg, Wnextp)


def _pred_call(accp, hwp, dinv, bg, Wp1p, bp1p, Wp2p, bp2p):
    BN = 2000
    grid = (N // BN,)

    def body(a_ref, hw_ref, dv_ref, bg_ref, w1_ref, b1_ref, w2_ref, b2_ref,
             out_ref):
        acc = a_ref[0] + a_ref[1] + a_ref[2] + a_ref[3] + hw_ref[...]
        g = jnp.maximum(acc * dv_ref[...] + bg_ref[...], 0.0)
        gp = jnp.concatenate(
            [g, jnp.zeros((BN, 128 - H), jnp.float32)], axis=1)
        t = jnp.tanh(jnp.dot(gp, w1_ref[...],
                             preferred_element_type=jnp.float32) + b1_ref[...])
        o = jnp.tanh(jnp.dot(t, w2_ref[...],
                             preferred_element_type=jnp.float32) + b2_ref[...])
        out_ref[...] = o[:, :1]

    return pl.pallas_call(
        body,
        grid=grid,
        in_specs=[
            pl.BlockSpec((4, BN, H), lambda i: (0, i, 0)),
            pl.BlockSpec((BN, H), lambda i: (i, 0)),
            pl.BlockSpec((BN, 1), lambda i: (i, 0)),
            pl.BlockSpec((1, H), lambda i: (0, 0)),
            pl.BlockSpec((128, 128), lambda i: (0, 0)),
            pl.BlockSpec((1, 128), lambda i: (0, 0)),
            pl.BlockSpec((128, 128), lambda i: (0, 0)),
            pl.BlockSpec((1, 128), lambda i: (0, 0)),
        ],
        out_specs=pl.BlockSpec((BN, 1), lambda i: (i, 0)),
        out_shape=jax.ShapeDtypeStruct((N, 1), jnp.float32),
    )(accp, hwp, dinv, bg, Wp1p, bp1p, Wp2p, bp2p)


def _pad2(w, shape):
    out = jnp.zeros(shape, jnp.float32)
    return out.at[: w.shape[0], : w.shape[1]].set(w)


def kernel(x, edge_index, S, R, We1, be1, We2, be2, Wg1, bg1, Wg2, bg2,
           Wp1, bp1, Wp2, bp2):
    npad = EPAD - E
    src3 = jnp.concatenate(
        [edge_index[0], jnp.zeros((npad,), jnp.int32)]).reshape(NTILES, CH, B)
    dst3 = jnp.concatenate(
        [edge_index[1], jnp.full((npad,), N, jnp.int32)]).reshape(NTILES, CH, B)

    zcol = jnp.zeros((NROWS, 16), jnp.float32)
    z32 = jnp.zeros((NROWS, H), jnp.float32)
    ones_col = jnp.ones((B, 16), jnp.float32)

    Rp = jnp.zeros((K, 128), jnp.float32).at[:, 6:9].set(R)
    xp = jnp.zeros((N, 128), jnp.float32).at[:, :6].set(x)
    We1p = _pad2(We1, (128, 128))
    be1p = jnp.zeros((1, 128), jnp.float32).at[0, :64].set(be1)
    We2p = _pad2(We2, (128, 128))
    be2p = jnp.zeros((1, 128), jnp.float32).at[0, :H].set(be2)
    Wg1p = _pad2(Wg1, (128, 128))
    Wg2p = _pad2(Wg2, (128, 128))
    Wp1p = _pad2(Wp1, (128, 128))
    bp1p = jnp.zeros((1, 128), jnp.float32).at[0, :H].set(bp1)
    Wp2p = _pad2(Wp2, (128, 128))
    bp2p = jnp.zeros((1, 128), jnp.float32).at[0, :1].set(bp2)
    bg1r = bg1.reshape(1, H)
    bg2r = bg2.reshape(1, H)

    degp = _deg_call(dst3, zcol, ones_col)[:, :N, :1]

    hw1 = _embed_call(S, Rp, xp, We1p, be1p, We2p, be2p, Wg1p)

    hwp1, dinv = _scale_call(hw1, degp)

    acc1 = _msg_call(hwp1, src3, dst3, z32)[:, :N]

    hwp2 = _layer_call(acc1, hwp1, dinv, bg1r, Wg2p)

    acc2 = _msg_call(hwp2, src3, dst3, z32)[:, :N]

    return _pred_call(acc2, hwp2, dinv, bg2r, Wp1p, bp1p, Wp2p, bp2p)


# speedup vs baseline: 1.0528x; 1.0528x over previous
import functools

import jax
import jax.numpy as jnp
from jax import lax
from jax.experimental import pallas as pl
from jax.experimental.pallas import tpu as pltpu
from jax.experimental.pallas import tpu_sc as plsc

N = 10000
E = 320000
K = 20000
H = 32
B = 128
NTILES = 32
NSUB = 16
CH = 80
EPT = CH * B
EPAD = NTILES * EPT
KBUF = 8
NROWS = 10112
RPS = NROWS // NSUB


def _sc_mesh():
    return plsc.VectorSubcoreMesh(core_axis_name="c", subcore_axis_name="s")


def _deg_call(dst3, zcol, ones_col):
    @functools.partial(
        pl.kernel,
        out_type=jax.ShapeDtypeStruct((2, NROWS, 16), jnp.float32),
        mesh=_sc_mesh(),
        scratch_types=[
            pltpu.VMEM((CH, B), jnp.int32),
            pltpu.VMEM((B, 16), jnp.float32),
            pltpu.VMEM_SHARED((NROWS, 16), jnp.float32),
        ],
        compiler_params=pltpu.CompilerParams(use_tc_tiling_on_sc=False),
    )
    def k(dst_hbm, z_hbm, ones_hbm, out_hbm, dst_all, ones_v, acc_sh):
        c = lax.axis_index("c")
        s = lax.axis_index("s")
        wid = c * NSUB + s
        pltpu.sync_copy(z_hbm.at[pl.ds(s * RPS, RPS)],
                        acc_sh.at[pl.ds(s * RPS, RPS)])
        pltpu.sync_copy(ones_hbm, ones_v)
        pltpu.sync_copy(dst_hbm.at[wid], dst_all)
        plsc.subcore_barrier()

        def body(g, _):
            pltpu.sync_copy(ones_v, acc_sh.at[dst_all.at[g]], add=True)
            return ()

        lax.fori_loop(0, CH, body, ())
        plsc.subcore_barrier()
        pltpu.sync_copy(acc_sh.at[pl.ds(s * RPS, RPS)],
                        out_hbm.at[c, pl.ds(s * RPS, RPS)])

    return k(dst3, zcol, ones_col)


def _msg_call(hwp, src3, dst3, z32):
    @functools.partial(
        pl.kernel,
        out_type=jax.ShapeDtypeStruct((2, NROWS, H), jnp.float32),
        mesh=_sc_mesh(),
        scratch_types=[
            pltpu.VMEM((CH, B), jnp.int32),
            pltpu.VMEM((CH, B), jnp.int32),
            [pltpu.VMEM((B, H), jnp.float32) for _ in range(KBUF)],
            pltpu.VMEM_SHARED((NROWS, H), jnp.float32),
            [pltpu.SemaphoreType.DMA for _ in range(KBUF)],
        ],
        compiler_params=pltpu.CompilerParams(use_tc_tiling_on_sc=False),
    )
    def k(hwp_hbm, src_hbm, dst_hbm, z_hbm, out_hbm,
          src_all, dst_all, bufs, acc_sh, sems):
        c = lax.axis_index("c")
        s = lax.axis_index("s")
        wid = c * NSUB + s
        pltpu.sync_copy(z_hbm.at[pl.ds(s * RPS, RPS)],
                        acc_sh.at[pl.ds(s * RPS, RPS)])
        pltpu.sync_copy(src_hbm.at[wid], src_all)
        pltpu.sync_copy(dst_hbm.at[wid], dst_all)
        plsc.subcore_barrier()

        def body(j, _):
            g0 = j * KBUF
            ds = [
                pltpu.async_copy(hwp_hbm.at[src_all.at[g0 + b]],
                                 bufs[b], sems[b])
                for b in range(KBUF)
            ]
            for b in range(KBUF):
                ds[b].wait()
                pltpu.sync_copy(bufs[b], acc_sh.at[dst_all.at[g0 + b]],
                                add=True)
            return ()

        lax.fori_loop(0, CH // KBUF, body, ())
        plsc.subcore_barrier()
        pltpu.sync_copy(acc_sh.at[pl.ds(s * RPS, RPS)],
                        out_hbm.at[c, pl.ds(s * RPS, RPS)])

    return k(hwp, src3, dst3, z32)


def _embed_call(S, Rp, xp, We1p, be1p, We2p, be2p, Wg1p):
    BN = 80
    grid = (N // BN,)

    def body(s_ref, rp_ref, xp_ref, w1_ref, b1_ref, w2_ref, b2_ref,
             wg_ref, hw_ref):
        sub = jnp.dot(s_ref[...], rp_ref[...],
                      preferred_element_type=jnp.float32)
        h = xp_ref[...] + sub
        t = jnp.tanh(jnp.dot(h, w1_ref[...],
                             preferred_element_type=jnp.float32) + b1_ref[...])
        h2 = jnp.tanh(jnp.dot(t, w2_ref[...],
                              preferred_element_type=jnp.float32) + b2_ref[...])
        hw1 = jnp.dot(h2, wg_ref[...], preferred_element_type=jnp.float32)
        hw_ref[...] = hw1[:, :H]

    return pl.pallas_call(
        body,
        grid=grid,
        in_specs=[
            pl.BlockSpec((BN, K), lambda i: (i, 0)),
            pl.BlockSpec((K, 128), lambda i: (0, 0)),
            pl.BlockSpec((BN, 128), lambda i: (i, 0)),
            pl.BlockSpec((128, 128), lambda i: (0, 0)),
            pl.BlockSpec((1, 128), lambda i: (0, 0)),
            pl.BlockSpec((128, 128), lambda i: (0, 0)),
            pl.BlockSpec((1, 128), lambda i: (0, 0)),
            pl.BlockSpec((128, 128), lambda i: (0, 0)),
        ],
        out_specs=pl.BlockSpec((BN, H), lambda i: (i, 0)),
        out_shape=jax.ShapeDtypeStruct((N, H), jnp.float32),
    )(S, Rp, xp, We1p, be1p, We2p, be2p, Wg1p)


def _scale_call(hw1, degp):
    BN = 2000
    grid = (N // BN,)

    def body(hw_ref, dg_ref, hwp_ref, dinv_ref):
        deg = dg_ref[0] + dg_ref[1] + 1.0
        dinv = lax.rsqrt(deg)
        hwp_ref[...] = hw_ref[...] * dinv
        dinv_ref[...] = dinv

    return pl.pallas_call(
        body,
        grid=grid,
        in_specs=[
            pl.BlockSpec((BN, H), lambda i: (i, 0)),
            pl.BlockSpec((2, BN, 1), lambda i: (0, i, 0)),
        ],
        out_specs=[
            pl.BlockSpec((BN, H), lambda i: (i, 0)),
            pl.BlockSpec((BN, 1), lambda i: (i, 0)),
        ],
        out_shape=[
            jax.ShapeDtypeStruct((N, H), jnp.float32),
            jax.ShapeDtypeStruct((N, 1), jnp.float32),
        ],
    )(hw1, degp)


def _layer_call(accp, hwp, dinv, bg, Wnextp):
    BN = 2000
    grid = (N // BN,)

    def body(a_ref, hw_ref, dv_ref, bg_ref, wn_ref, out_ref):
        acc = a_ref[0] + a_ref[1] + hw_ref[...]
        g = jnp.maximum(acc * dv_ref[...] + bg_ref[...], 0.0)
        gp = jnp.concatenate(
            [g, jnp.zeros((BN, 128 - H), jnp.float32)], axis=1)
        hw2 = jnp.dot(gp, wn_ref[...], preferred_element_type=jnp.float32)
        out_ref[...] = (hw2 * dv_ref[...])[:, :H]

    return pl.pallas_call(
        body,
        grid=grid,
        in_specs=[
            pl.BlockSpec((2, BN, H), lambda i: (0, i, 0)),
            pl.BlockSpec((BN, H), lambda i: (i, 0)),
            pl.BlockSpec((BN, 1), lambda i: (i, 0)),
            pl.BlockSpec((1, H), lambda i: (0, 0)),
            pl.BlockSpec((128, 128), lambda i: (0, 0)),
        ],
        out_specs=pl.BlockSpec((BN, H), lambda i: (i, 0)),
        out_shape=jax.ShapeDtypeStruct((N, H), jnp.float32),
    )(accp, hwp, dinv, bg, Wnextp)


def _pred_call(accp, hwp, dinv, bg, Wp1p, bp1p, Wp2p, bp2p):
    BN = 2000
    grid = (N // BN,)

    def body(a_ref, hw_ref, dv_ref, bg_ref, w1_ref, b1_ref, w2_ref, b2_ref,
             out_ref):
        acc = a_ref[0] + a_ref[1] + hw_ref[...]
        g = jnp.maximum(acc * dv_ref[...] + bg_ref[...], 0.0)
        gp = jnp.concatenate(
            [g, jnp.zeros((BN, 128 - H), jnp.float32)], axis=1)
        t = jnp.tanh(jnp.dot(gp, w1_ref[...],
                             preferred_element_type=jnp.float32) + b1_ref[...])
        o = jnp.tanh(jnp.dot(t, w2_ref[...],
                             preferred_element_type=jnp.float32) + b2_ref[...])
        out_ref[...] = o[:, :1]

    return pl.pallas_call(
        body,
        grid=grid,
        in_specs=[
            pl.BlockSpec((2, BN, H), lambda i: (0, i, 0)),
            pl.BlockSpec((BN, H), lambda i: (i, 0)),
            pl.BlockSpec((BN, 1), lambda i: (i, 0)),
            pl.BlockSpec((1, H), lambda i: (0, 0)),
            pl.BlockSpec((128, 128), lambda i: (0, 0)),
            pl.BlockSpec((1, 128), lambda i: (0, 0)),
            pl.BlockSpec((128, 128), lambda i: (0, 0)),
            pl.BlockSpec((1, 128), lambda i: (0, 0)),
        ],
        out_specs=pl.BlockSpec((BN, 1), lambda i: (i, 0)),
        out_shape=jax.ShapeDtypeStruct((N, 1), jnp.float32),
    )(accp, hwp, dinv, bg, Wp1p, bp1p, Wp2p, bp2p)


def _pad2(w, shape):
    out = jnp.zeros(shape, jnp.float32)
    return out.at[: w.shape[0], : w.shape[1]].set(w)


def kernel(x, edge_index, S, R, We1, be1, We2, be2, Wg1, bg1, Wg2, bg2,
           Wp1, bp1, Wp2, bp2):
    npad = EPAD - E
    src3 = jnp.concatenate(
        [edge_index[0], jnp.zeros((npad,), jnp.int32)]).reshape(NTILES, CH, B)
    dst3 = jnp.concatenate(
        [edge_index[1], jnp.full((npad,), N, jnp.int32)]).reshape(NTILES, CH, B)

    zcol = jnp.zeros((NROWS, 16), jnp.float32)
    z32 = jnp.zeros((NROWS, H), jnp.float32)
    ones_col = jnp.ones((B, 16), jnp.float32)

    Rp = jnp.zeros((K, 128), jnp.float32).at[:, 6:9].set(R)
    xp = jnp.zeros((N, 128), jnp.float32).at[:, :6].set(x)
    We1p = _pad2(We1, (128, 128))
    be1p = jnp.zeros((1, 128), jnp.float32).at[0, :64].set(be1)
    We2p = _pad2(We2, (128, 128))
    be2p = jnp.zeros((1, 128), jnp.float32).at[0, :H].set(be2)
    Wg1p = _pad2(Wg1, (128, 128))
    Wg2p = _pad2(Wg2, (128, 128))
    Wp1p = _pad2(Wp1, (128, 128))
    bp1p = jnp.zeros((1, 128), jnp.float32).at[0, :H].set(bp1)
    Wp2p = _pad2(Wp2, (128, 128))
    bp2p = jnp.zeros((1, 128), jnp.float32).at[0, :1].set(bp2)
    bg1r = bg1.reshape(1, H)
    bg2r = bg2.reshape(1, H)

    degp = _deg_call(dst3, zcol, ones_col)[:, :N, :1]

    hw1 = _embed_call(S, Rp, xp, We1p, be1p, We2p, be2p, Wg1p)

    hwp1, dinv = _scale_call(hw1, degp)

    acc1 = _msg_call(hwp1, src3, dst3, z32)[:, :N]

    hwp2 = _layer_call(acc1, hwp1, dinv, bg1r, Wg2p)

    acc2 = _msg_call(hwp2, src3, dst3, z32)[:, :N]

    return _pred_call(acc2, hwp2, dinv, bg2r, Wp1p, bp1p, Wp2p, bp2p)


# speedup vs baseline: 1.0730x; 1.0192x over previous
import functools

import jax
import jax.numpy as jnp
from jax import lax
from jax.experimental import pallas as pl
from jax.experimental.pallas import tpu as pltpu
from jax.experimental.pallas import tpu_sc as plsc

N = 10000
E = 320000
K = 20000
H = 32
B = 128
NTILES = 32
NSUB = 16
CH = 80
EPT = CH * B
EPAD = NTILES * EPT
KBUF = 8
NROWS = 10112
RPS = NROWS // NSUB


def _sc_mesh():
    return plsc.VectorSubcoreMesh(core_axis_name="c", subcore_axis_name="s")


def _deg_call(dst3, zcol, ones_col):
    @functools.partial(
        pl.kernel,
        out_type=jax.ShapeDtypeStruct((2, NROWS, 16), jnp.float32),
        mesh=_sc_mesh(),
        scratch_types=[
            pltpu.VMEM((CH, B), jnp.int32),
            pltpu.VMEM((B, 16), jnp.float32),
            pltpu.VMEM_SHARED((NROWS, 16), jnp.float32),
        ],
        compiler_params=pltpu.CompilerParams(use_tc_tiling_on_sc=False),
    )
    def k(dst_hbm, z_hbm, ones_hbm, out_hbm, dst_all, ones_v, acc_sh):
        c = lax.axis_index("c")
        s = lax.axis_index("s")
        wid = c * NSUB + s
        pltpu.sync_copy(z_hbm.at[pl.ds(s * RPS, RPS)],
                        acc_sh.at[pl.ds(s * RPS, RPS)])
        pltpu.sync_copy(ones_hbm, ones_v)
        pltpu.sync_copy(dst_hbm.at[wid], dst_all)
        plsc.subcore_barrier()

        def body(g, _):
            pltpu.sync_copy(ones_v, acc_sh.at[dst_all.at[g]], add=True)
            return ()

        lax.fori_loop(0, CH, body, ())
        plsc.subcore_barrier()
        pltpu.sync_copy(acc_sh.at[pl.ds(s * RPS, RPS)],
                        out_hbm.at[c, pl.ds(s * RPS, RPS)])

    return k(dst3, zcol, ones_col)


def _msg_call(hwp, src3, dst3, z32):
    @functools.partial(
        pl.kernel,
        out_type=jax.ShapeDtypeStruct((2, NROWS, H), jnp.float32),
        mesh=_sc_mesh(),
        scratch_types=[
            pltpu.VMEM((CH, B), jnp.int32),
            pltpu.VMEM((CH, B), jnp.int32),
            [pltpu.VMEM((B, H), jnp.float32) for _ in range(KBUF)],
            pltpu.VMEM_SHARED((NROWS, H), jnp.float32),
            [pltpu.SemaphoreType.DMA for _ in range(KBUF)],
            [pltpu.SemaphoreType.DMA for _ in range(KBUF)],
        ],
        compiler_params=pltpu.CompilerParams(use_tc_tiling_on_sc=False),
    )
    def k(hwp_hbm, src_hbm, dst_hbm, z_hbm, out_hbm,
          src_all, dst_all, bufs, acc_sh, gsems, ssems):
        c = lax.axis_index("c")
        s = lax.axis_index("s")
        wid = c * NSUB + s
        pltpu.sync_copy(z_hbm.at[pl.ds(s * RPS, RPS)],
                        acc_sh.at[pl.ds(s * RPS, RPS)])
        pltpu.sync_copy(src_hbm.at[wid], src_all)
        pltpu.sync_copy(dst_hbm.at[wid], dst_all)
        plsc.subcore_barrier()

        def drain_scatter(b):
            pltpu.make_async_copy(hwp_hbm.at[pl.ds(0, B)], bufs[b],
                                  ssems[b]).wait()

        def body(j, _):
            g0 = j * KBUF
            ds = []
            for b in range(KBUF):
                @pl.when(j > 0)
                def _():
                    drain_scatter(b)
                ds.append(pltpu.async_copy(
                    hwp_hbm.at[src_all.at[g0 + b]], bufs[b], gsems[b]))
            for b in range(KBUF):
                ds[b].wait()
                pltpu.async_copy(bufs[b], acc_sh.at[dst_all.at[g0 + b]],
                                 ssems[b], add=True)
            return ()

        lax.fori_loop(0, CH // KBUF, body, ())
        for b in range(KBUF):
            drain_scatter(b)
        plsc.subcore_barrier()
        pltpu.sync_copy(acc_sh.at[pl.ds(s * RPS, RPS)],
                        out_hbm.at[c, pl.ds(s * RPS, RPS)])

    return k(hwp, src3, dst3, z32)


def _embed_call(S, Rp, xp, We1p, be1p, We2p, be2p, Wg1p):
    BN = 80
    grid = (N // BN,)

    def body(s_ref, rp_ref, xp_ref, w1_ref, b1_ref, w2_ref, b2_ref,
             wg_ref, hw_ref):
        sub = jnp.dot(s_ref[...], rp_ref[...],
                      preferred_element_type=jnp.float32)
        h = xp_ref[...] + sub
        t = jnp.tanh(jnp.dot(h, w1_ref[...],
                             preferred_element_type=jnp.float32) + b1_ref[...])
        h2 = jnp.tanh(jnp.dot(t, w2_ref[...],
                              preferred_element_type=jnp.float32) + b2_ref[...])
        hw1 = jnp.dot(h2, wg_ref[...], preferred_element_type=jnp.float32)
        hw_ref[...] = hw1[:, :H]

    return pl.pallas_call(
        body,
        grid=grid,
        in_specs=[
            pl.BlockSpec((BN, K), lambda i: (i, 0)),
            pl.BlockSpec((K, 128), lambda i: (0, 0)),
            pl.BlockSpec((BN, 128), lambda i: (i, 0)),
            pl.BlockSpec((128, 128), lambda i: (0, 0)),
            pl.BlockSpec((1, 128), lambda i: (0, 0)),
            pl.BlockSpec((128, 128), lambda i: (0, 0)),
            pl.BlockSpec((1, 128), lambda i: (0, 0)),
            pl.BlockSpec((128, 128), lambda i: (0, 0)),
        ],
        out_specs=pl.BlockSpec((BN, H), lambda i: (i, 0)),
        out_shape=jax.ShapeDtypeStruct((N, H), jnp.float32),
    )(S, Rp, xp, We1p, be1p, We2p, be2p, Wg1p)


def _scale_call(hw1, degp):
    BN = 2000
    grid = (N // BN,)

    def body(hw_ref, dg_ref, hwp_ref, dinv_ref):
        deg = dg_ref[0] + dg_ref[1] + 1.0
        dinv = lax.rsqrt(deg)
        hwp_ref[...] = hw_ref[...] * dinv
        dinv_ref[...] = dinv

    return pl.pallas_call(
        body,
        grid=grid,
        in_specs=[
            pl.BlockSpec((BN, H), lambda i: (i, 0)),
            pl.BlockSpec((2, BN, 1), lambda i: (0, i, 0)),
        ],
        out_specs=[
            pl.BlockSpec((BN, H), lambda i: (i, 0)),
            pl.BlockSpec((BN, 1), lambda i: (i, 0)),
        ],
        out_shape=[
            jax.ShapeDtypeStruct((N, H), jnp.float32),
            jax.ShapeDtypeStruct((N, 1), jnp.float32),
        ],
    )(hw1, degp)


def _layer_call(accp, hwp, dinv, bg, Wnextp):
    BN = 2000
    grid = (N // BN,)

    def body(a_ref, hw_ref, dv_ref, bg_ref, wn_ref, out_ref):
        acc = a_ref[0] + a_ref[1] + hw_ref[...]
        g = jnp.maximum(acc * dv_ref[...] + bg_ref[...], 0.0)
        gp = jnp.concatenate(
            [g, jnp.zeros((BN, 128 - H), jnp.float32)], axis=1)
        hw2 = jnp.dot(gp, wn_ref[...], preferred_element_type=jnp.float32)
        out_ref[...] = (hw2 * dv_ref[...])[:, :H]

    return pl.pallas_call(
        body,
        grid=grid,
        in_specs=[
            pl.BlockSpec((2, BN, H), lambda i: (0, i, 0)),
            pl.BlockSpec((BN, H), lambda i: (i, 0)),
            pl.BlockSpec((BN, 1), lambda i: (i, 0)),
            pl.BlockSpec((1, H), lambda i: (0, 0)),
            pl.BlockSpec((128, 128), lambda i: (0, 0)),
        ],
        out_specs=pl.BlockSpec((BN, H), lambda i: (i, 0)),
        out_shape=jax.ShapeDtypeStruct((N, H), jnp.float32),
    )(accp, hwp, dinv, bg, Wnextp)


def _pred_call(accp, hwp, dinv, bg, Wp1p, bp1p, Wp2p, bp2p):
    BN = 2000
    grid = (N // BN,)

    def body(a_ref, hw_ref, dv_ref, bg_ref, w1_ref, b1_ref, w2_ref, b2_ref,
             out_ref):
        acc = a_ref[0] + a_ref[1] + hw_ref[...]
        g = jnp.maximum(acc * dv_ref[...] + bg_ref[...], 0.0)
        gp = jnp.concatenate(
            [g, jnp.zeros((BN, 128 - H), jnp.float32)], axis=1)
        t = jnp.tanh(jnp.dot(gp, w1_ref[...],
                             preferred_element_type=jnp.float32) + b1_ref[...])
        o = jnp.tanh(jnp.dot(t, w2_ref[...],
                             preferred_element_type=jnp.float32) + b2_ref[...])
        out_ref[...] = o[:, :1]

    return pl.pallas_call(
        body,
        grid=grid,
        in_specs=[
            pl.BlockSpec((2, BN, H), lambda i: (0, i, 0)),
            pl.BlockSpec((BN, H), lambda i: (i, 0)),
            pl.BlockSpec((BN, 1), lambda i: (i, 0)),
            pl.BlockSpec((1, H), lambda i: (0, 0)),
            pl.BlockSpec((128, 128), lambda i: (0, 0)),
            pl.BlockSpec((1, 128), lambda i: (0, 0)),
            pl.BlockSpec((128, 128), lambda i: (0, 0)),
            pl.BlockSpec((1, 128), lambda i: (0, 0)),
        ],
        out_specs=pl.BlockSpec((BN, 1), lambda i: (i, 0)),
        out_shape=jax.ShapeDtypeStruct((N, 1), jnp.float32),
    )(accp, hwp, dinv, bg, Wp1p, bp1p, Wp2p, bp2p)


def _pad2(w, shape):
    out = jnp.zeros(shape, jnp.float32)
    return out.at[: w.shape[0], : w.shape[1]].set(w)


def kernel(x, edge_index, S, R, We1, be1, We2, be2, Wg1, bg1, Wg2, bg2,
           Wp1, bp1, Wp2, bp2):
    npad = EPAD - E
    src3 = jnp.concatenate(
        [edge_index[0], jnp.zeros((npad,), jnp.int32)]).reshape(NTILES, CH, B)
    dst3 = jnp.concatenate(
        [edge_index[1], jnp.full((npad,), N, jnp.int32)]).reshape(NTILES, CH, B)

    zcol = jnp.zeros((NROWS, 16), jnp.float32)
    z32 = jnp.zeros((NROWS, H), jnp.float32)
    ones_col = jnp.ones((B, 16), jnp.float32)

    Rp = jnp.zeros((K, 128), jnp.float32).at[:, 6:9].set(R)
    xp = jnp.zeros((N, 128), jnp.float32).at[:, :6].set(x)
    We1p = _pad2(We1, (128, 128))
    be1p = jnp.zeros((1, 128), jnp.float32).at[0, :64].set(be1)
    We2p = _pad2(We2, (128, 128))
    be2p = jnp.zeros((1, 128), jnp.float32).at[0, :H].set(be2)
    Wg1p = _pad2(Wg1, (128, 128))
    Wg2p = _pad2(Wg2, (128, 128))
    Wp1p = _pad2(Wp1, (128, 128))
    bp1p = jnp.zeros((1, 128), jnp.float32).at[0, :H].set(bp1)
    Wp2p = _pad2(Wp2, (128, 128))
    bp2p = jnp.zeros((1, 128), jnp.float32).at[0, :1].set(bp2)
    bg1r = bg1.reshape(1, H)
    bg2r = bg2.reshape(1, H)

    degp = _deg_call(dst3, zcol, ones_col)[:, :N, :1]

    hw1 = _embed_call(S, Rp, xp, We1p, be1p, We2p, be2p, Wg1p)

    hwp1, dinv = _scale_call(hw1, degp)

    acc1 = _msg_call(hwp1, src3, dst3, z32)[:, :N]

    hwp2 = _layer_call(acc1, hwp1, dinv, bg1r, Wg2p)

    acc2 = _msg_call(hwp2, src3, dst3, z32)[:, :N]

    return _pred_call(acc2, hwp2, dinv, bg2r, Wp1p, bp1p, Wp2p, bp2p)


# speedup vs baseline: 1.4823x; 1.3814x over previous
import functools

import jax
import jax.numpy as jnp
from jax import lax
from jax.experimental import pallas as pl
from jax.experimental.pallas import tpu as pltpu
from jax.experimental.pallas import tpu_sc as plsc

N = 10000
E = 320000
K = 20000
H = 32
B = 128
NTILES = 32
NSUB = 16
CH = 80
EPT = CH * B
EPAD = NTILES * EPT
KBUF = 8
NROWS = 10112
RPS = NROWS // NSUB


def _sc_mesh():
    return plsc.VectorSubcoreMesh(core_axis_name="c", subcore_axis_name="s")


def _deg_call(dst3, zcol, ones_col):
    @functools.partial(
        pl.kernel,
        out_type=jax.ShapeDtypeStruct((2, NROWS, 16), jnp.float32),
        mesh=_sc_mesh(),
        scratch_types=[
            pltpu.VMEM((CH, B), jnp.int32),
            pltpu.VMEM((B, 16), jnp.float32),
            pltpu.VMEM_SHARED((NROWS, 16), jnp.float32),
        ],
        compiler_params=pltpu.CompilerParams(use_tc_tiling_on_sc=False),
    )
    def k(dst_hbm, z_hbm, ones_hbm, out_hbm, dst_all, ones_v, acc_sh):
        c = lax.axis_index("c")
        s = lax.axis_index("s")
        wid = c * NSUB + s
        pltpu.sync_copy(z_hbm.at[pl.ds(s * RPS, RPS)],
                        acc_sh.at[pl.ds(s * RPS, RPS)])
        pltpu.sync_copy(ones_hbm, ones_v)
        pltpu.sync_copy(dst_hbm.at[wid], dst_all)
        plsc.subcore_barrier()

        def body(g, _):
            pltpu.sync_copy(ones_v, acc_sh.at[dst_all.at[g]], add=True)
            return ()

        lax.fori_loop(0, CH, body, ())
        plsc.subcore_barrier()
        pltpu.sync_copy(acc_sh.at[pl.ds(s * RPS, RPS)],
                        out_hbm.at[c, pl.ds(s * RPS, RPS)])

    return k(dst3, zcol, ones_col)


def _msg_call(hwp, src3, dst3, z32):
    @functools.partial(
        pl.kernel,
        out_type=jax.ShapeDtypeStruct((2, NROWS, H), jnp.float32),
        mesh=_sc_mesh(),
        scratch_types=[
            pltpu.VMEM((CH, B), jnp.int32),
            pltpu.VMEM((CH, B), jnp.int32),
            [pltpu.VMEM((B, H), jnp.float32) for _ in range(KBUF)],
            pltpu.VMEM_SHARED((NROWS, H), jnp.float32),
            pltpu.VMEM_SHARED((NROWS, H), jnp.float32),
            [pltpu.SemaphoreType.DMA for _ in range(KBUF)],
            [pltpu.SemaphoreType.DMA for _ in range(KBUF)],
        ],
        compiler_params=pltpu.CompilerParams(use_tc_tiling_on_sc=False),
    )
    def k(hwp_hbm, src_hbm, dst_hbm, z_hbm, out_hbm,
          src_all, dst_all, bufs, acc_sh, hwp_sh, gsems, ssems):
        c = lax.axis_index("c")
        s = lax.axis_index("s")
        wid = c * NSUB + s
        pltpu.sync_copy(z_hbm.at[pl.ds(s * RPS, RPS)],
                        acc_sh.at[pl.ds(s * RPS, RPS)])
        pltpu.sync_copy(hwp_hbm.at[pl.ds(s * RPS, RPS)],
                        hwp_sh.at[pl.ds(s * RPS, RPS)])
        pltpu.sync_copy(src_hbm.at[wid], src_all)
        pltpu.sync_copy(dst_hbm.at[wid], dst_all)
        plsc.subcore_barrier()

        def drain_scatter(b):
            pltpu.make_async_copy(hwp_hbm.at[pl.ds(0, B)], bufs[b],
                                  ssems[b]).wait()

        def body(j, _):
            g0 = j * KBUF
            ds = []
            for b in range(KBUF):
                @pl.when(j > 0)
                def _():
                    drain_scatter(b)
                ds.append(pltpu.async_copy(
                    hwp_sh.at[src_all.at[g0 + b]], bufs[b], gsems[b]))
            for b in range(KBUF):
                ds[b].wait()
                pltpu.async_copy(bufs[b], acc_sh.at[dst_all.at[g0 + b]],
                                 ssems[b], add=True)
            return ()

        lax.fori_loop(0, CH // KBUF, body, ())
        for b in range(KBUF):
            drain_scatter(b)
        plsc.subcore_barrier()
        pltpu.sync_copy(acc_sh.at[pl.ds(s * RPS, RPS)],
                        out_hbm.at[c, pl.ds(s * RPS, RPS)])

    return k(hwp, src3, dst3, z32)


def _embed_call(S, Rp, xp, We1p, be1p, We2p, be2p, Wg1p):
    BN = 80
    grid = (N // BN,)

    def body(s_ref, rp_ref, xp_ref, w1_ref, b1_ref, w2_ref, b2_ref,
             wg_ref, hw_ref):
        sub = jnp.dot(s_ref[...], rp_ref[...],
                      preferred_element_type=jnp.float32)
        h = xp_ref[...] + sub
        t = jnp.tanh(jnp.dot(h, w1_ref[...],
                             preferred_element_type=jnp.float32) + b1_ref[...])
        h2 = jnp.tanh(jnp.dot(t, w2_ref[...],
                              preferred_element_type=jnp.float32) + b2_ref[...])
        hw1 = jnp.dot(h2, wg_ref[...], preferred_element_type=jnp.float32)
        hw_ref[...] = hw1[:, :H]

    return pl.pallas_call(
        body,
        grid=grid,
        in_specs=[
            pl.BlockSpec((BN, K), lambda i: (i, 0)),
            pl.BlockSpec((K, 128), lambda i: (0, 0)),
            pl.BlockSpec((BN, 128), lambda i: (i, 0)),
            pl.BlockSpec((128, 128), lambda i: (0, 0)),
            pl.BlockSpec((1, 128), lambda i: (0, 0)),
            pl.BlockSpec((128, 128), lambda i: (0, 0)),
            pl.BlockSpec((1, 128), lambda i: (0, 0)),
            pl.BlockSpec((128, 128), lambda i: (0, 0)),
        ],
        out_specs=pl.BlockSpec((BN, H), lambda i: (i, 0)),
        out_shape=jax.ShapeDtypeStruct((N, H), jnp.float32),
    )(S, Rp, xp, We1p, be1p, We2p, be2p, Wg1p)


def _scale_call(hw1, degp):
    BN = 2000
    grid = (N // BN,)

    def body(hw_ref, dg_ref, hwp_ref, dinv_ref):
        deg = dg_ref[0] + dg_ref[1] + 1.0
        dinv = lax.rsqrt(deg)
        hwp_ref[...] = hw_ref[...] * dinv
        dinv_ref[...] = dinv

    return pl.pallas_call(
        body,
        grid=grid,
        in_specs=[
            pl.BlockSpec((BN, H), lambda i: (i, 0)),
            pl.BlockSpec((2, BN, 1), lambda i: (0, i, 0)),
        ],
        out_specs=[
            pl.BlockSpec((BN, H), lambda i: (i, 0)),
            pl.BlockSpec((BN, 1), lambda i: (i, 0)),
        ],
        out_shape=[
            jax.ShapeDtypeStruct((N, H), jnp.float32),
            jax.ShapeDtypeStruct((N, 1), jnp.float32),
        ],
    )(hw1, degp)


def _layer_call(accp, hwp, dinv, bg, Wnextp):
    BN = 2000
    grid = (N // BN,)

    def body(a_ref, hw_ref, dv_ref, bg_ref, wn_ref, out_ref):
        acc = a_ref[0] + a_ref[1] + hw_ref[...]
        g = jnp.maximum(acc * dv_ref[...] + bg_ref[...], 0.0)
        gp = jnp.concatenate(
            [g, jnp.zeros((BN, 128 - H), jnp.float32)], axis=1)
        hw2 = jnp.dot(gp, wn_ref[...], preferred_element_type=jnp.float32)
        out_ref[...] = (hw2 * dv_ref[...])[:, :H]

    return pl.pallas_call(
        body,
        grid=grid,
        in_specs=[
            pl.BlockSpec((2, BN, H), lambda i: (0, i, 0)),
            pl.BlockSpec((BN, H), lambda i: (i, 0)),
            pl.BlockSpec((BN, 1), lambda i: (i, 0)),
            pl.BlockSpec((1, H), lambda i: (0, 0)),
            pl.BlockSpec((128, 128), lambda i: (0, 0)),
        ],
        out_specs=pl.BlockSpec((BN, H), lambda i: (i, 0)),
        out_shape=jax.ShapeDtypeStruct((N, H), jnp.float32),
    )(accp, hwp, dinv, bg, Wnextp)


def _pred_call(accp, hwp, dinv, bg, Wp1p, bp1p, Wp2p, bp2p):
    BN = 2000
    grid = (N // BN,)

    def body(a_ref, hw_ref, dv_ref, bg_ref, w1_ref, b1_ref, w2_ref, b2_ref,
             out_ref):
        acc = a_ref[0] + a_ref[1] + hw_ref[...]
        g = jnp.maximum(acc * dv_ref[...] + bg_ref[...], 0.0)
        gp = jnp.concatenate(
            [g, jnp.zeros((BN, 128 - H), jnp.float32)], axis=1)
        t = jnp.tanh(jnp.dot(gp, w1_ref[...],
                             preferred_element_type=jnp.float32) + b1_ref[...])
        o = jnp.tanh(jnp.dot(t, w2_ref[...],
                             preferred_element_type=jnp.float32) + b2_ref[...])
        out_ref[...] = o[:, :1]

    return pl.pallas_call(
        body,
        grid=grid,
        in_specs=[
            pl.BlockSpec((2, BN, H), lambda i: (0, i, 0)),
            pl.BlockSpec((BN, H), lambda i: (i, 0)),
            pl.BlockSpec((BN, 1), lambda i: (i, 0)),
            pl.BlockSpec((1, H), lambda i: (0, 0)),
            pl.BlockSpec((128, 128), lambda i: (0, 0)),
            pl.BlockSpec((1, 128), lambda i: (0, 0)),
            pl.BlockSpec((128, 128), lambda i: (0, 0)),
            pl.BlockSpec((1, 128), lambda i: (0, 0)),
        ],
        out_specs=pl.BlockSpec((BN, 1), lambda i: (i, 0)),
        out_shape=jax.ShapeDtypeStruct((N, 1), jnp.float32),
    )(accp, hwp, dinv, bg, Wp1p, bp1p, Wp2p, bp2p)


def _pad2(w, shape):
    out = jnp.zeros(shape, jnp.float32)
    return out.at[: w.shape[0], : w.shape[1]].set(w)


def kernel(x, edge_index, S, R, We1, be1, We2, be2, Wg1, bg1, Wg2, bg2,
           Wp1, bp1, Wp2, bp2):
    npad = EPAD - E
    src3 = jnp.concatenate(
        [edge_index[0], jnp.zeros((npad,), jnp.int32)]).reshape(NTILES, CH, B)
    dst3 = jnp.concatenate(
        [edge_index[1], jnp.full((npad,), N, jnp.int32)]).reshape(NTILES, CH, B)

    zcol = jnp.zeros((NROWS, 16), jnp.float32)
    z32 = jnp.zeros((NROWS, H), jnp.float32)
    ones_col = jnp.ones((B, 16), jnp.float32)

    Rp = jnp.zeros((K, 128), jnp.float32).at[:, 6:9].set(R)
    xp = jnp.zeros((N, 128), jnp.float32).at[:, :6].set(x)
    We1p = _pad2(We1, (128, 128))
    be1p = jnp.zeros((1, 128), jnp.float32).at[0, :64].set(be1)
    We2p = _pad2(We2, (128, 128))
    be2p = jnp.zeros((1, 128), jnp.float32).at[0, :H].set(be2)
    Wg1p = _pad2(Wg1, (128, 128))
    Wg2p = _pad2(Wg2, (128, 128))
    Wp1p = _pad2(Wp1, (128, 128))
    bp1p = jnp.zeros((1, 128), jnp.float32).at[0, :H].set(bp1)
    Wp2p = _pad2(Wp2, (128, 128))
    bp2p = jnp.zeros((1, 128), jnp.float32).at[0, :1].set(bp2)
    bg1r = bg1.reshape(1, H)
    bg2r = bg2.reshape(1, H)

    degp = _deg_call(dst3, zcol, ones_col)[:, :N, :1]

    hw1 = _embed_call(S, Rp, xp, We1p, be1p, We2p, be2p, Wg1p)

    hwp1, dinv = _scale_call(hw1, degp)

    hwp1pad = jnp.zeros((NROWS, H), jnp.float32).at[:N].set(hwp1)
    acc1 = _msg_call(hwp1pad, src3, dst3, z32)[:, :N]

    hwp2 = _layer_call(acc1, hwp1, dinv, bg1r, Wg2p)

    hwp2pad = jnp.zeros((NROWS, H), jnp.float32).at[:N].set(hwp2)
    acc2 = _msg_call(hwp2pad, src3, dst3, z32)[:, :N]

    return _pred_call(acc2, hwp2, dinv, bg2r, Wp1p, bp1p, Wp2p, bp2p)


# speedup vs baseline: 1.4934x; 1.0075x over previous
import functools

import jax
import jax.numpy as jnp
from jax import lax
from jax.experimental import pallas as pl
from jax.experimental.pallas import tpu as pltpu
from jax.experimental.pallas import tpu_sc as plsc

N = 10000
E = 320000
K = 20000
H = 32
B = 128
NTILES = 32
NSUB = 16
CH = 80
EPT = CH * B
EPAD = NTILES * EPT
KBUF = 8
NROWS = 10112
RPS = NROWS // NSUB


def _sc_mesh():
    return plsc.VectorSubcoreMesh(core_axis_name="c", subcore_axis_name="s")


def _deg_call(dst3, zcol, ones_col):
    @functools.partial(
        pl.kernel,
        out_type=jax.ShapeDtypeStruct((2, NROWS, 16), jnp.float32),
        mesh=_sc_mesh(),
        scratch_types=[
            pltpu.VMEM((CH, B), jnp.int32),
            pltpu.VMEM((B, 16), jnp.float32),
            pltpu.VMEM_SHARED((NROWS, 16), jnp.float32),
        ],
        compiler_params=pltpu.CompilerParams(use_tc_tiling_on_sc=False),
    )
    def k(dst_hbm, z_hbm, ones_hbm, out_hbm, dst_all, ones_v, acc_sh):
        c = lax.axis_index("c")
        s = lax.axis_index("s")
        wid = c * NSUB + s
        pltpu.sync_copy(z_hbm.at[pl.ds(s * RPS, RPS)],
                        acc_sh.at[pl.ds(s * RPS, RPS)])
        pltpu.sync_copy(ones_hbm, ones_v)
        pltpu.sync_copy(dst_hbm.at[wid], dst_all)
        plsc.subcore_barrier()

        def body(g, _):
            pltpu.sync_copy(ones_v, acc_sh.at[dst_all.at[g]], add=True)
            return ()

        lax.fori_loop(0, CH, body, ())
        plsc.subcore_barrier()
        pltpu.sync_copy(acc_sh.at[pl.ds(s * RPS, RPS)],
                        out_hbm.at[c, pl.ds(s * RPS, RPS)])

    return k(dst3, zcol, ones_col)


def _msg_call(hwp, src3, dst3, z32):
    @functools.partial(
        pl.kernel,
        out_type=jax.ShapeDtypeStruct((2, NROWS, H), jnp.float32),
        mesh=_sc_mesh(),
        scratch_types=[
            pltpu.VMEM((CH, B), jnp.int32),
            pltpu.VMEM((CH, B), jnp.int32),
            [pltpu.VMEM((B, H), jnp.float32) for _ in range(KBUF)],
            pltpu.VMEM_SHARED((NROWS, H), jnp.float32),
            pltpu.VMEM_SHARED((NROWS, H), jnp.float32),
            [pltpu.SemaphoreType.DMA for _ in range(KBUF)],
            [pltpu.SemaphoreType.DMA for _ in range(KBUF)],
        ],
        compiler_params=pltpu.CompilerParams(use_tc_tiling_on_sc=False),
    )
    def k(hwp_hbm, src_hbm, dst_hbm, z_hbm, out_hbm,
          src_all, dst_all, bufs, acc_sh, hwp_sh, gsems, ssems):
        c = lax.axis_index("c")
        s = lax.axis_index("s")
        wid = c * NSUB + s
        pltpu.sync_copy(z_hbm.at[pl.ds(s * RPS, RPS)],
                        acc_sh.at[pl.ds(s * RPS, RPS)])
        pltpu.sync_copy(hwp_hbm.at[pl.ds(s * RPS, RPS)],
                        hwp_sh.at[pl.ds(s * RPS, RPS)])
        pltpu.sync_copy(src_hbm.at[wid], src_all)
        pltpu.sync_copy(dst_hbm.at[wid], dst_all)
        plsc.subcore_barrier()

        def drain_scatter(b):
            pltpu.make_async_copy(hwp_hbm.at[pl.ds(0, B)], bufs[b],
                                  ssems[b]).wait()

        def body(j, _):
            g0 = j * KBUF
            ds = []
            for b in range(KBUF):
                @pl.when(j > 0)
                def _():
                    drain_scatter(b)
                ds.append(pltpu.async_copy(
                    hwp_sh.at[src_all.at[g0 + b]], bufs[b], gsems[b]))
            for b in range(KBUF):
                ds[b].wait()
                pltpu.async_copy(bufs[b], acc_sh.at[dst_all.at[g0 + b]],
                                 ssems[b], add=True)
            return ()

        lax.fori_loop(0, CH // KBUF, body, ())
        for b in range(KBUF):
            drain_scatter(b)
        plsc.subcore_barrier()
        pltpu.sync_copy(acc_sh.at[pl.ds(s * RPS, RPS)],
                        out_hbm.at[c, pl.ds(s * RPS, RPS)])

    return k(hwp, src3, dst3, z32)


def _embed_call(S, Rp, xp, We1p, be1p, We2p, be2p, Wg1p):
    BN = 80
    grid = (N // BN,)

    def body(s_ref, rp_ref, xp_ref, w1_ref, b1_ref, w2_ref, b2_ref,
             wg_ref, hw_ref):
        sub = jnp.dot(s_ref[...].astype(jnp.bfloat16), rp_ref[...],
                      preferred_element_type=jnp.float32)
        h = xp_ref[...] + sub
        t = jnp.tanh(jnp.dot(h, w1_ref[...],
                             preferred_element_type=jnp.float32) + b1_ref[...])
        h2 = jnp.tanh(jnp.dot(t, w2_ref[...],
                              preferred_element_type=jnp.float32) + b2_ref[...])
        hw1 = jnp.dot(h2, wg_ref[...], preferred_element_type=jnp.float32)
        hw_ref[...] = hw1[:, :H]

    return pl.pallas_call(
        body,
        grid=grid,
        in_specs=[
            pl.BlockSpec((BN, K), lambda i: (i, 0)),
            pl.BlockSpec((K, 128), lambda i: (0, 0)),
            pl.BlockSpec((BN, 128), lambda i: (i, 0)),
            pl.BlockSpec((128, 128), lambda i: (0, 0)),
            pl.BlockSpec((1, 128), lambda i: (0, 0)),
            pl.BlockSpec((128, 128), lambda i: (0, 0)),
            pl.BlockSpec((1, 128), lambda i: (0, 0)),
            pl.BlockSpec((128, 128), lambda i: (0, 0)),
        ],
        out_specs=pl.BlockSpec((BN, H), lambda i: (i, 0)),
        out_shape=jax.ShapeDtypeStruct((N, H), jnp.float32),
    )(S, Rp, xp, We1p, be1p, We2p, be2p, Wg1p)


def _scale_call(hw1, degp):
    BN = 2000
    grid = (N // BN,)

    def body(hw_ref, dg_ref, hwp_ref, dinv_ref):
        deg = dg_ref[0] + dg_ref[1] + 1.0
        dinv = lax.rsqrt(deg)
        hwp_ref[...] = hw_ref[...] * dinv
        dinv_ref[...] = dinv

    return pl.pallas_call(
        body,
        grid=grid,
        in_specs=[
            pl.BlockSpec((BN, H), lambda i: (i, 0)),
            pl.BlockSpec((2, BN, 1), lambda i: (0, i, 0)),
        ],
        out_specs=[
            pl.BlockSpec((BN, H), lambda i: (i, 0)),
            pl.BlockSpec((BN, 1), lambda i: (i, 0)),
        ],
        out_shape=[
            jax.ShapeDtypeStruct((N, H), jnp.float32),
            jax.ShapeDtypeStruct((N, 1), jnp.float32),
        ],
    )(hw1, degp)


def _layer_call(accp, hwp, dinv, bg, Wnextp):
    BN = 2000
    grid = (N // BN,)

    def body(a_ref, hw_ref, dv_ref, bg_ref, wn_ref, out_ref):
        acc = a_ref[0] + a_ref[1] + hw_ref[...]
        g = jnp.maximum(acc * dv_ref[...] + bg_ref[...], 0.0)
        gp = jnp.concatenate(
            [g, jnp.zeros((BN, 128 - H), jnp.float32)], axis=1)
        hw2 = jnp.dot(gp, wn_ref[...], preferred_element_type=jnp.float32)
        out_ref[...] = (hw2 * dv_ref[...])[:, :H]

    return pl.pallas_call(
        body,
        grid=grid,
        in_specs=[
            pl.BlockSpec((2, BN, H), lambda i: (0, i, 0)),
            pl.BlockSpec((BN, H), lambda i: (i, 0)),
            pl.BlockSpec((BN, 1), lambda i: (i, 0)),
            pl.BlockSpec((1, H), lambda i: (0, 0)),
            pl.BlockSpec((128, 128), lambda i: (0, 0)),
        ],
        out_specs=pl.BlockSpec((BN, H), lambda i: (i, 0)),
        out_shape=jax.ShapeDtypeStruct((N, H), jnp.float32),
    )(accp, hwp, dinv, bg, Wnextp)


def _pred_call(accp, hwp, dinv, bg, Wp1p, bp1p, Wp2p, bp2p):
    BN = 2000
    grid = (N // BN,)

    def body(a_ref, hw_ref, dv_ref, bg_ref, w1_ref, b1_ref, w2_ref, b2_ref,
             out_ref):
        acc = a_ref[0] + a_ref[1] + hw_ref[...]
        g = jnp.maximum(acc * dv_ref[...] + bg_ref[...], 0.0)
        gp = jnp.concatenate(
            [g, jnp.zeros((BN, 128 - H), jnp.float32)], axis=1)
        t = jnp.tanh(jnp.dot(gp, w1_ref[...],
                             preferred_element_type=jnp.float32) + b1_ref[...])
        o = jnp.tanh(jnp.dot(t, w2_ref[...],
                             preferred_element_type=jnp.float32) + b2_ref[...])
        out_ref[...] = o[:, :1]

    return pl.pallas_call(
        body,
        grid=grid,
        in_specs=[
            pl.BlockSpec((2, BN, H), lambda i: (0, i, 0)),
            pl.BlockSpec((BN, H), lambda i: (i, 0)),
            pl.BlockSpec((BN, 1), lambda i: (i, 0)),
            pl.BlockSpec((1, H), lambda i: (0, 0)),
            pl.BlockSpec((128, 128), lambda i: (0, 0)),
            pl.BlockSpec((1, 128), lambda i: (0, 0)),
            pl.BlockSpec((128, 128), lambda i: (0, 0)),
            pl.BlockSpec((1, 128), lambda i: (0, 0)),
        ],
        out_specs=pl.BlockSpec((BN, 1), lambda i: (i, 0)),
        out_shape=jax.ShapeDtypeStruct((N, 1), jnp.float32),
    )(accp, hwp, dinv, bg, Wp1p, bp1p, Wp2p, bp2p)


def _pad2(w, shape):
    out = jnp.zeros(shape, jnp.float32)
    return out.at[: w.shape[0], : w.shape[1]].set(w)


def kernel(x, edge_index, S, R, We1, be1, We2, be2, Wg1, bg1, Wg2, bg2,
           Wp1, bp1, Wp2, bp2):
    npad = EPAD - E
    src3 = jnp.concatenate(
        [edge_index[0], jnp.zeros((npad,), jnp.int32)]).reshape(NTILES, CH, B)
    dst3 = jnp.concatenate(
        [edge_index[1], jnp.full((npad,), N, jnp.int32)]).reshape(NTILES, CH, B)

    zcol = jnp.zeros((NROWS, 16), jnp.float32)
    z32 = jnp.zeros((NROWS, H), jnp.float32)
    ones_col = jnp.ones((B, 16), jnp.float32)

    Rp = jnp.zeros((K, 128), jnp.float32).at[:, 6:9].set(R).astype(jnp.bfloat16)
    xp = jnp.zeros((N, 128), jnp.float32).at[:, :6].set(x)
    We1p = _pad2(We1, (128, 128))
    be1p = jnp.zeros((1, 128), jnp.float32).at[0, :64].set(be1)
    We2p = _pad2(We2, (128, 128))
    be2p = jnp.zeros((1, 128), jnp.float32).at[0, :H].set(be2)
    Wg1p = _pad2(Wg1, (128, 128))
    Wg2p = _pad2(Wg2, (128, 128))
    Wp1p = _pad2(Wp1, (128, 128))
    bp1p = jnp.zeros((1, 128), jnp.float32).at[0, :H].set(bp1)
    Wp2p = _pad2(Wp2, (128, 128))
    bp2p = jnp.zeros((1, 128), jnp.float32).at[0, :1].set(bp2)
    bg1r = bg1.reshape(1, H)
    bg2r = bg2.reshape(1, H)

    degp = _deg_call(dst3, zcol, ones_col)[:, :N, :1]

    hw1 = _embed_call(S, Rp, xp, We1p, be1p, We2p, be2p, Wg1p)

    hwp1, dinv = _scale_call(hw1, degp)

    hwp1pad = jnp.zeros((NROWS, H), jnp.float32).at[:N].set(hwp1)
    acc1 = _msg_call(hwp1pad, src3, dst3, z32)[:, :N]

    hwp2 = _layer_call(acc1, hwp1, dinv, bg1r, Wg2p)

    hwp2pad = jnp.zeros((NROWS, H), jnp.float32).at[:N].set(hwp2)
    acc2 = _msg_call(hwp2pad, src3, dst3, z32)[:, :N]

    return _pred_call(acc2, hwp2, dinv, bg2r, Wp1p, bp1p, Wp2p, bp2p)


# speedup vs baseline: 1.6127x; 1.0799x over previous
import functools

import jax
import jax.numpy as jnp
from jax import lax
from jax.experimental import pallas as pl
from jax.experimental.pallas import tpu as pltpu
from jax.experimental.pallas import tpu_sc as plsc

N = 10000
E = 320000
K = 20000
H = 32
B = 128
NTILES = 32
NSUB = 16
CH = 80
EPT = CH * B
EPAD = NTILES * EPT
KBUF = 8
NROWS = 10112
RPS = NROWS // NSUB


def _sc_mesh():
    return plsc.VectorSubcoreMesh(core_axis_name="c", subcore_axis_name="s")


def _deg_call(dst3, zcol, ones_col):
    @functools.partial(
        pl.kernel,
        out_type=jax.ShapeDtypeStruct((2, NROWS, 16), jnp.float32),
        mesh=_sc_mesh(),
        scratch_types=[
            pltpu.VMEM((CH, B), jnp.int32),
            pltpu.VMEM((B, 16), jnp.float32),
            pltpu.VMEM_SHARED((NROWS, 16), jnp.float32),
        ],
        compiler_params=pltpu.CompilerParams(use_tc_tiling_on_sc=False),
    )
    def k(dst_hbm, z_hbm, ones_hbm, out_hbm, dst_all, ones_v, acc_sh):
        c = lax.axis_index("c")
        s = lax.axis_index("s")
        wid = c * NSUB + s
        pltpu.sync_copy(z_hbm.at[pl.ds(s * RPS, RPS)],
                        acc_sh.at[pl.ds(s * RPS, RPS)])
        pltpu.sync_copy(ones_hbm, ones_v)
        pltpu.sync_copy(dst_hbm.at[wid], dst_all)
        plsc.subcore_barrier()

        def body(g, _):
            pltpu.sync_copy(ones_v, acc_sh.at[dst_all.at[g]], add=True)
            return ()

        lax.fori_loop(0, CH, body, ())
        plsc.subcore_barrier()
        pltpu.sync_copy(acc_sh.at[pl.ds(s * RPS, RPS)],
                        out_hbm.at[c, pl.ds(s * RPS, RPS)])

    return k(dst3, zcol, ones_col)


def _msg_call(hwp, src3, dst3, z32):
    @functools.partial(
        pl.kernel,
        out_type=jax.ShapeDtypeStruct((2, NROWS, H), jnp.float32),
        mesh=_sc_mesh(),
        scratch_types=[
            pltpu.VMEM((CH, B), jnp.int32),
            pltpu.VMEM((CH, B), jnp.int32),
            [pltpu.VMEM((B, H), jnp.float32) for _ in range(KBUF)],
            pltpu.VMEM_SHARED((NROWS, H), jnp.float32),
            pltpu.VMEM_SHARED((NROWS, H), jnp.float32),
            [pltpu.SemaphoreType.DMA for _ in range(KBUF)],
            [pltpu.SemaphoreType.DMA for _ in range(KBUF)],
        ],
        compiler_params=pltpu.CompilerParams(use_tc_tiling_on_sc=False),
    )
    def k(hwp_hbm, src_hbm, dst_hbm, z_hbm, out_hbm,
          src_all, dst_all, bufs, acc_sh, hwp_sh, gsems, ssems):
        c = lax.axis_index("c")
        s = lax.axis_index("s")
        wid = c * NSUB + s
        pltpu.sync_copy(z_hbm.at[pl.ds(s * RPS, RPS)],
                        acc_sh.at[pl.ds(s * RPS, RPS)])
        pltpu.sync_copy(hwp_hbm.at[pl.ds(s * RPS, RPS)],
                        hwp_sh.at[pl.ds(s * RPS, RPS)])
        pltpu.sync_copy(src_hbm.at[wid], src_all)
        pltpu.sync_copy(dst_hbm.at[wid], dst_all)
        plsc.subcore_barrier()

        def drain_scatter(b):
            pltpu.make_async_copy(hwp_hbm.at[pl.ds(0, B)], bufs[b],
                                  ssems[b]).wait()

        def body(j, _):
            g0 = j * KBUF
            ds = []
            for b in range(KBUF):
                @pl.when(j > 0)
                def _():
                    drain_scatter(b)
                ds.append(pltpu.async_copy(
                    hwp_sh.at[src_all.at[g0 + b]], bufs[b], gsems[b]))
            for b in range(KBUF):
                ds[b].wait()
                pltpu.async_copy(bufs[b], acc_sh.at[dst_all.at[g0 + b]],
                                 ssems[b], add=True)
            return ()

        lax.fori_loop(0, CH // KBUF, body, ())
        for b in range(KBUF):
            drain_scatter(b)
        plsc.subcore_barrier()
        pltpu.sync_copy(acc_sh.at[pl.ds(s * RPS, RPS)],
                        out_hbm.at[c, pl.ds(s * RPS, RPS)])

    return k(hwp, src3, dst3, z32)


def _embed_call(S, Rp, xp, We1p, be1p, We2p, be2p, Wg1p):
    BN = 200
    grid = (N // BN,)

    def body(s_ref, rp_ref, xp_ref, w1_ref, b1_ref, w2_ref, b2_ref,
             wg_ref, hw_ref):
        sub = jnp.dot(s_ref[...].astype(jnp.bfloat16), rp_ref[...],
                      preferred_element_type=jnp.float32)
        h = xp_ref[...] + sub
        t = jnp.tanh(jnp.dot(h, w1_ref[...],
                             preferred_element_type=jnp.float32) + b1_ref[...])
        h2 = jnp.tanh(jnp.dot(t, w2_ref[...],
                              preferred_element_type=jnp.float32) + b2_ref[...])
        hw1 = jnp.dot(h2, wg_ref[...], preferred_element_type=jnp.float32)
        hw_ref[...] = hw1[:, :H]

    return pl.pallas_call(
        body,
        grid=grid,
        in_specs=[
            pl.BlockSpec((BN, K), lambda i: (i, 0)),
            pl.BlockSpec((K, 128), lambda i: (0, 0)),
            pl.BlockSpec((BN, 128), lambda i: (i, 0)),
            pl.BlockSpec((128, 128), lambda i: (0, 0)),
            pl.BlockSpec((1, 128), lambda i: (0, 0)),
            pl.BlockSpec((128, 128), lambda i: (0, 0)),
            pl.BlockSpec((1, 128), lambda i: (0, 0)),
            pl.BlockSpec((128, 128), lambda i: (0, 0)),
        ],
        out_specs=pl.BlockSpec((BN, H), lambda i: (i, 0)),
        out_shape=jax.ShapeDtypeStruct((N, H), jnp.float32),
    )(S, Rp, xp, We1p, be1p, We2p, be2p, Wg1p)


def _scale_call(hw1, degp):
    BN = 2000
    grid = (N // BN,)

    def body(hw_ref, dg_ref, hwp_ref, dinv_ref):
        deg = dg_ref[0] + dg_ref[1] + 1.0
        dinv = lax.rsqrt(deg)
        hwp_ref[...] = hw_ref[...] * dinv
        dinv_ref[...] = dinv

    return pl.pallas_call(
        body,
        grid=grid,
        in_specs=[
            pl.BlockSpec((BN, H), lambda i: (i, 0)),
            pl.BlockSpec((2, BN, 1), lambda i: (0, i, 0)),
        ],
        out_specs=[
            pl.BlockSpec((BN, H), lambda i: (i, 0)),
            pl.BlockSpec((BN, 1), lambda i: (i, 0)),
        ],
        out_shape=[
            jax.ShapeDtypeStruct((N, H), jnp.float32),
            jax.ShapeDtypeStruct((N, 1), jnp.float32),
        ],
    )(hw1, degp)


def _layer_call(accp, hwp, dinv, bg, Wnextp):
    BN = 2000
    grid = (N // BN,)

    def body(a_ref, hw_ref, dv_ref, bg_ref, wn_ref, out_ref):
        acc = a_ref[0] + a_ref[1] + hw_ref[...]
        g = jnp.maximum(acc * dv_ref[...] + bg_ref[...], 0.0)
        gp = jnp.concatenate(
            [g, jnp.zeros((BN, 128 - H), jnp.float32)], axis=1)
        hw2 = jnp.dot(gp, wn_ref[...], preferred_element_type=jnp.float32)
        out_ref[...] = (hw2 * dv_ref[...])[:, :H]

    return pl.pallas_call(
        body,
        grid=grid,
        in_specs=[
            pl.BlockSpec((2, BN, H), lambda i: (0, i, 0)),
            pl.BlockSpec((BN, H), lambda i: (i, 0)),
            pl.BlockSpec((BN, 1), lambda i: (i, 0)),
            pl.BlockSpec((1, H), lambda i: (0, 0)),
            pl.BlockSpec((128, 128), lambda i: (0, 0)),
        ],
        out_specs=pl.BlockSpec((BN, H), lambda i: (i, 0)),
        out_shape=jax.ShapeDtypeStruct((N, H), jnp.float32),
    )(accp, hwp, dinv, bg, Wnextp)


def _pred_call(accp, hwp, dinv, bg, Wp1p, bp1p, Wp2p, bp2p):
    BN = 2000
    grid = (N // BN,)

    def body(a_ref, hw_ref, dv_ref, bg_ref, w1_ref, b1_ref, w2_ref, b2_ref,
             out_ref):
        acc = a_ref[0] + a_ref[1] + hw_ref[...]
        g = jnp.maximum(acc * dv_ref[...] + bg_ref[...], 0.0)
        gp = jnp.concatenate(
            [g, jnp.zeros((BN, 128 - H), jnp.float32)], axis=1)
        t = jnp.tanh(jnp.dot(gp, w1_ref[...],
                             preferred_element_type=jnp.float32) + b1_ref[...])
        o = jnp.tanh(jnp.dot(t, w2_ref[...],
                             preferred_element_type=jnp.float32) + b2_ref[...])
        out_ref[...] = o[:, :1]

    return pl.pallas_call(
        body,
        grid=grid,
        in_specs=[
            pl.BlockSpec((2, BN, H), lambda i: (0, i, 0)),
            pl.BlockSpec((BN, H), lambda i: (i, 0)),
            pl.BlockSpec((BN, 1), lambda i: (i, 0)),
            pl.BlockSpec((1, H), lambda i: (0, 0)),
            pl.BlockSpec((128, 128), lambda i: (0, 0)),
            pl.BlockSpec((1, 128), lambda i: (0, 0)),
            pl.BlockSpec((128, 128), lambda i: (0, 0)),
            pl.BlockSpec((1, 128), lambda i: (0, 0)),
        ],
        out_specs=pl.BlockSpec((BN, 1), lambda i: (i, 0)),
        out_shape=jax.ShapeDtypeStruct((N, 1), jnp.float32),
    )(accp, hwp, dinv, bg, Wp1p, bp1p, Wp2p, bp2p)


def _pad2(w, shape):
    out = jnp.zeros(shape, jnp.float32)
    return out.at[: w.shape[0], : w.shape[1]].set(w)


def kernel(x, edge_index, S, R, We1, be1, We2, be2, Wg1, bg1, Wg2, bg2,
           Wp1, bp1, Wp2, bp2):
    npad = EPAD - E
    src3 = jnp.concatenate(
        [edge_index[0], jnp.zeros((npad,), jnp.int32)]).reshape(NTILES, CH, B)
    dst3 = jnp.concatenate(
        [edge_index[1], jnp.full((npad,), N, jnp.int32)]).reshape(NTILES, CH, B)

    zcol = jnp.zeros((NROWS, 16), jnp.float32)
    z32 = jnp.zeros((NROWS, H), jnp.float32)
    ones_col = jnp.ones((B, 16), jnp.float32)

    Rp = jnp.zeros((K, 128), jnp.float32).at[:, 6:9].set(R).astype(jnp.bfloat16)
    xp = jnp.zeros((N, 128), jnp.float32).at[:, :6].set(x)
    We1p = _pad2(We1, (128, 128))
    be1p = jnp.zeros((1, 128), jnp.float32).at[0, :64].set(be1)
    We2p = _pad2(We2, (128, 128))
    be2p = jnp.zeros((1, 128), jnp.float32).at[0, :H].set(be2)
    Wg1p = _pad2(Wg1, (128, 128))
    Wg2p = _pad2(Wg2, (128, 128))
    Wp1p = _pad2(Wp1, (128, 128))
    bp1p = jnp.zeros((1, 128), jnp.float32).at[0, :H].set(bp1)
    Wp2p = _pad2(Wp2, (128, 128))
    bp2p = jnp.zeros((1, 128), jnp.float32).at[0, :1].set(bp2)
    bg1r = bg1.reshape(1, H)
    bg2r = bg2.reshape(1, H)

    degp = _deg_call(dst3, zcol, ones_col)[:, :N, :1]

    hw1 = _embed_call(S, Rp, xp, We1p, be1p, We2p, be2p, Wg1p)

    hwp1, dinv = _scale_call(hw1, degp)

    hwp1pad = jnp.zeros((NROWS, H), jnp.float32).at[:N].set(hwp1)
    acc1 = _msg_call(hwp1pad, src3, dst3, z32)[:, :N]

    hwp2 = _layer_call(acc1, hwp1, dinv, bg1r, Wg2p)

    hwp2pad = jnp.zeros((NROWS, H), jnp.float32).at[:N].set(hwp2)
    acc2 = _msg_call(hwp2pad, src3, dst3, z32)[:, :N]

    return _pred_call(acc2, hwp2, dinv, bg2r, Wp1p, bp1p, Wp2p, bp2p)
